# Initial kernel scaffold; baseline (speedup 1.0000x reference)
import functools

import jax
import jax.numpy as jnp
from jax import lax
from jax.experimental import pallas as pl
from jax.experimental.pallas import tpu as pltpu
from jax.experimental.pallas import tpu_sc as plsc

N_NODES = 10000
N_EDGES = 320000
D_FEAT = 128

NUM_CORES = 2
NUM_SUBCORES = 16
NUM_WORKERS = NUM_CORES * NUM_SUBCORES
EDGES_PER_WORKER = N_EDGES // NUM_WORKERS
LANES = 16
STEPS = EDGES_PER_WORKER // LANES


def _tc_table_body(x_ref, w_ref, b_ref, out_ref):
    out_ref[...] = (
        jnp.dot(x_ref[...], w_ref[...], preferred_element_type=jnp.float32)
        + b_ref[...]
    )


def _make_table(x, w12, b2):
    return pl.pallas_call(
        _tc_table_body,
        out_shape=jax.ShapeDtypeStruct((N_NODES, 2), jnp.float32),
    )(x, w12, b2)


@functools.partial(
    pl.kernel,
    mesh=plsc.VectorSubcoreMesh(core_axis_name="c", subcore_axis_name="s"),
    out_type=jax.ShapeDtypeStruct((N_EDGES,), jnp.float32),
    scratch_types=[
        pltpu.VMEM((N_NODES, 2), jnp.float32),
        pltpu.VMEM((EDGES_PER_WORKER,), jnp.int32),
        pltpu.VMEM((EDGES_PER_WORKER,), jnp.int32),
        pltpu.VMEM((EDGES_PER_WORKER,), jnp.float32),
    ],
)
def _sc_edge_scores(table_hbm, src_hbm, dst_hbm, out_hbm, t_v, s_v, d_v, o_v):
    wid = lax.axis_index("s") * NUM_CORES + lax.axis_index("c")
    base = wid * EDGES_PER_WORKER

    pltpu.sync_copy(table_hbm, t_v)
    pltpu.sync_copy(src_hbm.at[pl.ds(base, EDGES_PER_WORKER)], s_v)
    pltpu.sync_copy(dst_hbm.at[pl.ds(base, EDGES_PER_WORKER)], d_v)

    col0 = jnp.zeros((LANES,), jnp.int32)
    col1 = jnp.ones((LANES,), jnp.int32)

    def body(i, _):
        off = i * LANES
        si = s_v[pl.ds(off, LANES)]
        di = d_v[pl.ds(off, LANES)]
        g1 = plsc.load_gather(t_v, [si, col0])
        g2 = plsc.load_gather(t_v, [di, col1])
        o_v[pl.ds(off, LANES)] = g1 + g2
        return _

    lax.fori_loop(0, STEPS, body, None)

    pltpu.sync_copy(o_v, out_hbm.at[pl.ds(base, EDGES_PER_WORKER)])


def kernel(x, edge_index, W_weight, W_bias):
    w12 = jnp.transpose(W_weight.reshape(2, D_FEAT))
    b2 = jnp.stack([W_bias[0], jnp.zeros((), jnp.float32)]).reshape(1, 2)
    table = _make_table(x, w12, b2)

    edge_index = edge_index.astype(jnp.int32)
    src = edge_index[0]
    dst = edge_index[1]
    scores = _sc_edge_scores(table, src, dst)
    return scores.reshape(N_EDGES, 1)


# speedup vs baseline: 27.0121x; 27.0121x over previous
import functools

import jax
import jax.numpy as jnp
from jax import lax
from jax.experimental import pallas as pl
from jax.experimental.pallas import tpu as pltpu
from jax.experimental.pallas import tpu_sc as plsc

N_NODES = 10000
N_EDGES = 320000
D_FEAT = 128

NUM_CORES = 2
NUM_SUBCORES = 16
NUM_WORKERS = NUM_CORES * NUM_SUBCORES
EDGES_PER_WORKER = N_EDGES // NUM_WORKERS
LANES = 16
STEPS = EDGES_PER_WORKER // LANES


def _tc_table_body(x_ref, w_ref, b_ref, out_ref):
    out_ref[...] = (
        jnp.dot(x_ref[...], w_ref[...], preferred_element_type=jnp.float32)
        + b_ref[...]
    )


def _make_table(x, w12, b2):
    return pl.pallas_call(
        _tc_table_body,
        out_shape=jax.ShapeDtypeStruct((N_NODES, 2), jnp.float32),
    )(x, w12, b2)


@functools.partial(
    pl.kernel,
    mesh=plsc.VectorSubcoreMesh(core_axis_name="c", subcore_axis_name="s"),
    out_type=jax.ShapeDtypeStruct((N_EDGES,), jnp.float32),
    compiler_params=pltpu.CompilerParams(needs_layout_passes=False),
    scratch_types=[
        pltpu.VMEM((N_NODES,), jnp.float32),
        pltpu.VMEM((N_NODES,), jnp.float32),
        pltpu.VMEM((EDGES_PER_WORKER,), jnp.int32),
        pltpu.VMEM((EDGES_PER_WORKER,), jnp.int32),
        pltpu.VMEM((EDGES_PER_WORKER,), jnp.float32),
    ],
)
def _sc_edge_scores(t1_hbm, t2_hbm, src_hbm, dst_hbm, out_hbm,
                    t1_v, t2_v, s_v, d_v, o_v):
    wid = lax.axis_index("s") * NUM_CORES + lax.axis_index("c")
    base = wid * EDGES_PER_WORKER

    pltpu.sync_copy(t1_hbm, t1_v)
    pltpu.sync_copy(t2_hbm, t2_v)
    pltpu.sync_copy(src_hbm.at[pl.ds(base, EDGES_PER_WORKER)], s_v)
    pltpu.sync_copy(dst_hbm.at[pl.ds(base, EDGES_PER_WORKER)], d_v)

    def body(i, _):
        off = i * LANES
        si = s_v[pl.ds(off, LANES)]
        di = d_v[pl.ds(off, LANES)]
        g1 = plsc.load_gather(t1_v, [si])
        g2 = plsc.load_gather(t2_v, [di])
        o_v[pl.ds(off, LANES)] = g1 + g2
        return _

    lax.fori_loop(0, STEPS, body, None)

    pltpu.sync_copy(o_v, out_hbm.at[pl.ds(base, EDGES_PER_WORKER)])


def kernel(x, edge_index, W_weight, W_bias):
    w12 = jnp.transpose(W_weight.reshape(2, D_FEAT))
    b2 = jnp.stack([W_bias[0], jnp.zeros((), jnp.float32)]).reshape(1, 2)
    table = _make_table(x, w12, b2)
    t1 = table[:, 0]
    t2 = table[:, 1]

    edge_index = edge_index.astype(jnp.int32)
    src = edge_index[0]
    dst = edge_index[1]
    scores = _sc_edge_scores(t1, t2, src, dst)
    return scores.reshape(N_EDGES, 1)


# speedup vs baseline: 40.3572x; 1.4940x over previous
import functools

import jax
import jax.numpy as jnp
from jax import lax
from jax.experimental import pallas as pl
from jax.experimental.pallas import tpu as pltpu
from jax.experimental.pallas import tpu_sc as plsc

N_NODES = 10000
N_EDGES = 320000
D_FEAT = 128

NUM_CORES = 2
NUM_SUBCORES = 16
NUM_WORKERS = NUM_CORES * NUM_SUBCORES
EDGES_PER_WORKER = N_EDGES // NUM_WORKERS
LANES = 16
UNROLL = 5
STEPS = EDGES_PER_WORKER // (LANES * UNROLL)


def _tc_table_body(x_ref, w_ref, b_ref, out_ref):
    out_ref[...] = (
        lax.dot_general(
            w_ref[...], x_ref[...],
            (((1,), (1,)), ((), ())),
            preferred_element_type=jnp.float32,
        )
        + b_ref[...]
    )


def _make_table(x, w2r, b2):
    return pl.pallas_call(
        _tc_table_body,
        out_shape=jax.ShapeDtypeStruct((2, N_NODES), jnp.float32),
    )(x, w2r, b2)


@functools.partial(
    pl.kernel,
    mesh=plsc.VectorSubcoreMesh(core_axis_name="c", subcore_axis_name="s"),
    out_type=jax.ShapeDtypeStruct((N_EDGES,), jnp.float32),
    compiler_params=pltpu.CompilerParams(
        needs_layout_passes=False, use_tc_tiling_on_sc=False
    ),
    scratch_types=[
        pltpu.VMEM((N_NODES,), jnp.float32),
        pltpu.VMEM((N_NODES,), jnp.float32),
        pltpu.VMEM((EDGES_PER_WORKER,), jnp.int32),
        pltpu.VMEM((EDGES_PER_WORKER,), jnp.int32),
        pltpu.VMEM((EDGES_PER_WORKER,), jnp.float32),
        pltpu.SemaphoreType.DMA,
        pltpu.SemaphoreType.DMA,
        pltpu.SemaphoreType.DMA,
        pltpu.SemaphoreType.DMA,
    ],
)
def _sc_edge_scores(tab_hbm, ei_hbm, out_hbm,
                    t1_v, t2_v, s_v, d_v, o_v, sm1, sm2, sm3, sm4):
    wid = lax.axis_index("s") * NUM_CORES + lax.axis_index("c")
    base = wid * EDGES_PER_WORKER

    c1 = pltpu.async_copy(tab_hbm.at[0], t1_v, sm1)
    c2 = pltpu.async_copy(tab_hbm.at[1], t2_v, sm2)
    c3 = pltpu.async_copy(ei_hbm.at[0, pl.ds(base, EDGES_PER_WORKER)], s_v, sm3)
    c4 = pltpu.async_copy(ei_hbm.at[1, pl.ds(base, EDGES_PER_WORKER)], d_v, sm4)
    c1.wait()
    c2.wait()
    c3.wait()
    c4.wait()

    def body(i, _):
        for j in range(UNROLL):
            off = (i * UNROLL + j) * LANES
            si = s_v[pl.ds(off, LANES)]
            di = d_v[pl.ds(off, LANES)]
            g1 = plsc.load_gather(t1_v, [si])
            g2 = plsc.load_gather(t2_v, [di])
            o_v[pl.ds(off, LANES)] = g1 + g2
        return _

    lax.fori_loop(0, STEPS, body, None)

    pltpu.sync_copy(o_v, out_hbm.at[pl.ds(base, EDGES_PER_WORKER)])


def kernel(x, edge_index, W_weight, W_bias):
    w2r = W_weight.reshape(2, D_FEAT)
    b2 = jnp.stack([W_bias[0], jnp.zeros((), jnp.float32)]).reshape(2, 1)
    table = _make_table(x, w2r, b2)

    scores = _sc_edge_scores(table, edge_index.astype(jnp.int32))
    return scores.reshape(N_EDGES, 1)


# speedup vs baseline: 42.6559x; 1.0570x over previous
import functools

import jax
import jax.numpy as jnp
from jax import lax
from jax.experimental import pallas as pl
from jax.experimental.pallas import tpu as pltpu
from jax.experimental.pallas import tpu_sc as plsc

N_NODES = 10000
N_EDGES = 320000
D_FEAT = 128

NUM_CORES = 2
NUM_SUBCORES = 16
NUM_WORKERS = NUM_CORES * NUM_SUBCORES
EDGES_PER_WORKER = N_EDGES // NUM_WORKERS
LANES = 16
UNROLL = 5
STEPS = EDGES_PER_WORKER // (LANES * UNROLL)


def _tc_table_body(x_ref, w_ref, b_ref, out_ref):
    out_ref[...] = (
        lax.dot_general(
            w_ref[...], x_ref[...],
            (((1,), (1,)), ((), ())),
            preferred_element_type=jnp.float32,
        )
        + b_ref[...]
    )


def _make_table(x, w2r, b2):
    return pl.pallas_call(
        _tc_table_body,
        out_shape=jax.ShapeDtypeStruct((2, N_NODES), jnp.float32),
    )(x, w2r, b2)


@functools.partial(
    pl.kernel,
    mesh=plsc.VectorSubcoreMesh(core_axis_name="c", subcore_axis_name="s"),
    out_type=jax.ShapeDtypeStruct((1, N_EDGES), jnp.float32),
    compiler_params=pltpu.CompilerParams(
        needs_layout_passes=False, use_tc_tiling_on_sc=False
    ),
    scratch_types=[
        pltpu.VMEM((N_NODES,), jnp.float32),
        pltpu.VMEM((N_NODES,), jnp.float32),
        pltpu.VMEM((EDGES_PER_WORKER,), jnp.int32),
        pltpu.VMEM((EDGES_PER_WORKER,), jnp.int32),
        pltpu.VMEM((EDGES_PER_WORKER,), jnp.float32),
        pltpu.SemaphoreType.DMA,
        pltpu.SemaphoreType.DMA,
        pltpu.SemaphoreType.DMA,
        pltpu.SemaphoreType.DMA,
    ],
)
def _sc_edge_scores(tab_hbm, ei_hbm, out_hbm,
                    t1_v, t2_v, s_v, d_v, o_v, sm1, sm2, sm3, sm4):
    wid = lax.axis_index("s") * NUM_CORES + lax.axis_index("c")
    base = wid * EDGES_PER_WORKER

    c1 = pltpu.async_copy(tab_hbm.at[0], t1_v, sm1)
    c2 = pltpu.async_copy(tab_hbm.at[1], t2_v, sm2)
    c3 = pltpu.async_copy(ei_hbm.at[0, pl.ds(base, EDGES_PER_WORKER)], s_v, sm3)
    c4 = pltpu.async_copy(ei_hbm.at[1, pl.ds(base, EDGES_PER_WORKER)], d_v, sm4)
    c1.wait()
    c2.wait()
    c3.wait()
    c4.wait()

    @plsc.parallel_loop(0, EDGES_PER_WORKER, LANES, unroll=UNROLL)
    def _(off):
        si = s_v[pl.ds(off, LANES)]
        di = d_v[pl.ds(off, LANES)]
        g1 = plsc.load_gather(t1_v, [si])
        g2 = plsc.load_gather(t2_v, [di])
        o_v[pl.ds(off, LANES)] = g1 + g2

    pltpu.sync_copy(o_v, out_hbm.at[0, pl.ds(base, EDGES_PER_WORKER)])


def kernel(x, edge_index, W_weight, W_bias):
    w2r = W_weight.reshape(2, D_FEAT)
    b2 = jnp.stack([W_bias[0], jnp.zeros((), jnp.float32)]).reshape(2, 1)
    table = _make_table(x, w2r, b2)

    scores = _sc_edge_scores(table, edge_index.astype(jnp.int32))
    return scores.reshape(N_EDGES, 1)


# speedup vs baseline: 42.8157x; 1.0037x over previous
import functools

import jax
import jax.numpy as jnp
from jax import lax
from jax.experimental import pallas as pl
from jax.experimental.pallas import tpu as pltpu
from jax.experimental.pallas import tpu_sc as plsc

N_NODES = 10000
N_EDGES = 320000
D_FEAT = 128

NUM_CORES = 2
NUM_SUBCORES = 16
NUM_WORKERS = NUM_CORES * NUM_SUBCORES
EDGES_PER_WORKER = N_EDGES // NUM_WORKERS
LANES = 16
UNROLL = 5
STEPS = EDGES_PER_WORKER // (LANES * UNROLL)


def _tc_table_body(x_ref, w_ref, b_ref, out_ref):
    nt = (((1,), (1,)), ((), ()))
    s1 = lax.dot_general(w_ref[:, :D_FEAT], x_ref[...], nt,
                         preferred_element_type=jnp.float32)
    s2 = lax.dot_general(w_ref[:, D_FEAT:], x_ref[...], nt,
                         preferred_element_type=jnp.float32)
    out_ref[...] = jnp.concatenate([s1 + b_ref[0], s2], axis=0)


def _make_table(x, w, b):
    return pl.pallas_call(
        _tc_table_body,
        out_shape=jax.ShapeDtypeStruct((2, N_NODES), jnp.float32),
        in_specs=[
            pl.BlockSpec(memory_space=pltpu.VMEM),
            pl.BlockSpec(memory_space=pltpu.VMEM),
            pl.BlockSpec(memory_space=pltpu.SMEM),
        ],
    )(x, w, b)


@functools.partial(
    pl.kernel,
    mesh=plsc.VectorSubcoreMesh(core_axis_name="c", subcore_axis_name="s"),
    out_type=jax.ShapeDtypeStruct((1, N_EDGES), jnp.float32),
    compiler_params=pltpu.CompilerParams(
        needs_layout_passes=False, use_tc_tiling_on_sc=False
    ),
    scratch_types=[
        pltpu.VMEM((N_NODES,), jnp.float32),
        pltpu.VMEM((N_NODES,), jnp.float32),
        pltpu.VMEM((EDGES_PER_WORKER,), jnp.int32),
        pltpu.VMEM((EDGES_PER_WORKER,), jnp.int32),
        pltpu.VMEM((EDGES_PER_WORKER,), jnp.float32),
        pltpu.SemaphoreType.DMA,
        pltpu.SemaphoreType.DMA,
        pltpu.SemaphoreType.DMA,
        pltpu.SemaphoreType.DMA,
    ],
)
def _sc_edge_scores(tab_hbm, ei_hbm, out_hbm,
                    t1_v, t2_v, s_v, d_v, o_v, sm1, sm2, sm3, sm4):
    wid = lax.axis_index("s") * NUM_CORES + lax.axis_index("c")
    base = wid * EDGES_PER_WORKER

    c1 = pltpu.async_copy(tab_hbm.at[0], t1_v, sm1)
    c2 = pltpu.async_copy(tab_hbm.at[1], t2_v, sm2)
    c3 = pltpu.async_copy(ei_hbm.at[0, pl.ds(base, EDGES_PER_WORKER)], s_v, sm3)
    c4 = pltpu.async_copy(ei_hbm.at[1, pl.ds(base, EDGES_PER_WORKER)], d_v, sm4)
    c1.wait()
    c2.wait()
    c3.wait()
    c4.wait()

    @plsc.parallel_loop(0, EDGES_PER_WORKER, LANES, unroll=UNROLL)
    def _(off):
        si = s_v[pl.ds(off, LANES)]
        di = d_v[pl.ds(off, LANES)]
        g1 = plsc.load_gather(t1_v, [si])
        g2 = plsc.load_gather(t2_v, [di])
        o_v[pl.ds(off, LANES)] = g1 + g2

    pltpu.sync_copy(o_v, out_hbm.at[0, pl.ds(base, EDGES_PER_WORKER)])


def kernel(x, edge_index, W_weight, W_bias):
    table = _make_table(x, W_weight, W_bias)

    scores = _sc_edge_scores(table, edge_index.astype(jnp.int32))
    return scores.reshape(N_EDGES, 1)


# speedup vs baseline: 49.2043x; 1.1492x over previous
import functools

import jax
import jax.numpy as jnp
from jax import lax
from jax.experimental import pallas as pl
from jax.experimental.pallas import tpu as pltpu
from jax.experimental.pallas import tpu_sc as plsc

N_NODES = 10000
N_EDGES = 320000
D_FEAT = 128

NUM_CORES = 2
NUM_SUBCORES = 16
NUM_WORKERS = NUM_CORES * NUM_SUBCORES
EDGES_PER_WORKER = N_EDGES // NUM_WORKERS
LANES = 16
UNROLL = 5
WINDOW = EDGES_PER_WORKER + 112


def _tc_table_body(x_ref, w_ref, b_ref, t1_ref, t2_ref):
    nt = (((1,), (1,)), ((), ()))
    t1_ref[...] = lax.dot_general(w_ref[:, :D_FEAT], x_ref[...], nt,
                                  preferred_element_type=jnp.float32) + b_ref[0]
    t2_ref[...] = lax.dot_general(w_ref[:, D_FEAT:], x_ref[...], nt,
                                  preferred_element_type=jnp.float32)


def _make_tables(x, w, b):
    return pl.pallas_call(
        _tc_table_body,
        out_shape=[
            jax.ShapeDtypeStruct((1, N_NODES), jnp.float32),
            jax.ShapeDtypeStruct((1, N_NODES), jnp.float32),
        ],
        in_specs=[
            pl.BlockSpec(memory_space=pltpu.VMEM),
            pl.BlockSpec(memory_space=pltpu.VMEM),
            pl.BlockSpec(memory_space=pltpu.SMEM),
        ],
    )(x, w, b)


@functools.partial(
    pl.kernel,
    mesh=plsc.VectorSubcoreMesh(core_axis_name="c", subcore_axis_name="s"),
    out_type=jax.ShapeDtypeStruct((N_EDGES,), jnp.float32),
    compiler_params=pltpu.CompilerParams(needs_layout_passes=False),
    scratch_types=[
        pltpu.VMEM((N_NODES,), jnp.float32),
        pltpu.VMEM((N_NODES,), jnp.float32),
        pltpu.VMEM((2, WINDOW), jnp.int32),
        pltpu.VMEM((EDGES_PER_WORKER,), jnp.float32),
        pltpu.SemaphoreType.DMA,
        pltpu.SemaphoreType.DMA,
        pltpu.SemaphoreType.DMA,
    ],
)
def _sc_edge_scores(t1_hbm, t2_hbm, ei_hbm, out_hbm,
                    t1_v, t2_v, sd_v, o_v, sm1, sm2, sm3):
    wid = lax.axis_index("s") * NUM_CORES + lax.axis_index("c")
    base = wid * EDGES_PER_WORKER
    base_al = (base // 128) * 128
    delta = base - base_al

    c1 = pltpu.async_copy(t1_hbm.at[0], t1_v, sm1)
    c2 = pltpu.async_copy(t2_hbm.at[0], t2_v, sm2)
    c3 = pltpu.async_copy(ei_hbm.at[:, pl.ds(base_al, WINDOW)], sd_v, sm3)
    c1.wait()
    c2.wait()
    c3.wait()

    @plsc.parallel_loop(0, EDGES_PER_WORKER, LANES, unroll=UNROLL)
    def _(off):
        si = sd_v[0, pl.ds(delta + off, LANES)]
        di = sd_v[1, pl.ds(delta + off, LANES)]
        g1 = plsc.load_gather(t1_v, [si])
        g2 = plsc.load_gather(t2_v, [di])
        o_v[pl.ds(off, LANES)] = g1 + g2

    pltpu.sync_copy(o_v, out_hbm.at[pl.ds(base, EDGES_PER_WORKER)])


def kernel(x, edge_index, W_weight, W_bias):
    t1, t2 = _make_tables(x, W_weight, W_bias)
    scores = _sc_edge_scores(t1, t2, edge_index.astype(jnp.int32))
    return scores.reshape(N_EDGES, 1)
